# Initial kernel scaffold; baseline (speedup 1.0000x reference)
import jax
import jax.numpy as jnp
from jax.experimental import pallas as pl


def kernel(graph_state, graph_action, edge_index, W1, b1, W2, b2, W3, b3, L1_W, L1_b, L2_W, L2_b, W4, b4, W5, b5, W6, b6, L3_W, L3_b, L4_W, L4_b):
    raise NotImplementedError("write your pallas kernel here")


# speedup vs baseline: 37.5980x; 37.5980x over previous
import functools

import jax
import jax.numpy as jnp
from jax import lax
from jax.experimental import pallas as pl
from jax.experimental.pallas import tpu as pltpu
from jax.experimental.pallas import tpu_sc as plsc

N = 10000
E = 320000
NUM_AGENT = 100
STATE_DIM = 128

NC = 2
NS = 16
NW = NC * NS

N_PAD = 10240
KC = 16
CHUNK = KC * 128
E_PAD = 327680
CH = E_PAD // CHUNK
CPT = CH // NW
RPZ = N_PAD // NS

_mesh = plsc.VectorSubcoreMesh(core_axis_name="c", subcore_axis_name="s")
_sc_params = pltpu.CompilerParams(use_tc_tiling_on_sc=False)
F1 = 40
FD = 8


def _make_prop(F, name):
    @functools.partial(
        pl.kernel,
        out_type=jax.ShapeDtypeStruct((NC, N_PAD, F), jnp.float32),
        mesh=_mesh,
        scratch_types=[
            pltpu.VMEM((KC, 128), jnp.int32),
            pltpu.VMEM((KC, 128), jnp.int32),
            pltpu.VMEM((128, F), jnp.float32),
            pltpu.VMEM_SHARED((N_PAD, F), jnp.float32),
            pltpu.SemaphoreType.DMA,
        ],
        name=name,
        compiler_params=_sc_params,
    )
    def prop(xs_hbm, row_hbm, col_hbm, zeros_hbm, out_hbm,
             row_v, col_v, buf, acc, sem):
        cid = lax.axis_index("c")
        sid = lax.axis_index("s")
        w = cid * NS + sid
        pltpu.sync_copy(zeros_hbm.at[pl.ds(sid * RPZ, RPZ)],
                        acc.at[pl.ds(sid * RPZ, RPZ)])
        plsc.subcore_barrier()

        def chunk_body(i, carry):
            t = w * CPT + i
            pltpu.sync_copy(row_hbm.at[t], row_v)
            pltpu.sync_copy(col_hbm.at[t], col_v)
            for j in range(KC):
                pltpu.async_copy(xs_hbm.at[row_v.at[j]], buf, sem).wait()
                pltpu.sync_copy(buf, acc.at[col_v.at[j]], add=True)
            return carry

        lax.fori_loop(0, CPT, chunk_body, 0)
        plsc.subcore_barrier()
        pltpu.sync_copy(acc.at[pl.ds(sid * RPZ, RPZ)],
                        out_hbm.at[cid, pl.ds(sid * RPZ, RPZ)])

    return prop


_prop36 = _make_prop(F1, "sc_prop40")
_prop8 = _make_prop(8, "sc_prop8")


@functools.partial(
    pl.kernel,
    out_type=jax.ShapeDtypeStruct((NC, N_PAD, FD), jnp.float32),
    mesh=_mesh,
    scratch_types=[
        pltpu.VMEM((KC, 128), jnp.int32),
        pltpu.VMEM((128, FD), jnp.float32),
        pltpu.VMEM_SHARED((N_PAD, FD), jnp.float32),
    ],
    name="sc_deg",
    compiler_params=_sc_params,
)
def _deg(col_hbm, ones_hbm, zeros_hbm, out_hbm, col_v, ones_v, acc):
    cid = lax.axis_index("c")
    sid = lax.axis_index("s")
    w = cid * NS + sid
    pltpu.sync_copy(ones_hbm, ones_v)
    pltpu.sync_copy(zeros_hbm.at[pl.ds(sid * RPZ, RPZ)],
                    acc.at[pl.ds(sid * RPZ, RPZ)])
    plsc.subcore_barrier()

    def chunk_body(i, carry):
        t = w * CPT + i
        pltpu.sync_copy(col_hbm.at[t], col_v)
        for j in range(KC):
            pltpu.sync_copy(ones_v, acc.at[col_v.at[j]], add=True)
        return carry

    lax.fori_loop(0, CPT, chunk_body, 0)
    plsc.subcore_barrier()
    pltpu.sync_copy(acc.at[pl.ds(sid * RPZ, RPZ)],
                    out_hbm.at[cid, pl.ds(sid * RPZ, RPZ)])


def _tc1_body(s_ref, a_ref, w14_ref, degp_ref, xs_ref, dinv_ref):
    deg = degp_ref[0, :, 0:1] + degp_ref[1, :, 0:1] + 1.0
    dinv = lax.rsqrt(deg)
    u = jnp.dot(s_ref[...], w14_ref[...], preferred_element_type=jnp.float32)
    za = a_ref[...] * dinv
    xs_ref[...] = jnp.concatenate(
        [u * dinv, za, jnp.zeros((N_PAD, F1 - 33), jnp.float32)], axis=1)
    dinv_ref[...] = dinv


def _tc2_body(p_ref, xs_ref, dinv_ref, w2_ref, w5_ref, w3_ref, w6_ref,
              b1_ref, b2_ref, b4_ref, b5_ref, ys_ref):
    dinv = dinv_ref[...]
    h = (p_ref[0] + p_ref[1] + xs_ref[...]) * dinv
    za = h[:, 32:33]
    w3 = w3_ref[...]
    w6 = w6_ref[...]

    def branch(sl, b_s, w_a, b_a, w):
        xs_part = jax.nn.relu(h[:, sl] + b_s)
        xa_part = jax.nn.relu(
            jnp.dot(za, w_a, preferred_element_type=jnp.float32) + b_a)
        return (jnp.dot(xs_part, w[0:16], preferred_element_type=jnp.float32)
                + jnp.dot(xa_part, w[16:32], preferred_element_type=jnp.float32))

    y1 = branch(slice(0, 16), b1_ref[...], w2_ref[...], b2_ref[...], w3)
    y2 = branch(slice(16, 32), b4_ref[...], w5_ref[...], b5_ref[...], w6)
    y = jnp.concatenate([y1, y2, jnp.zeros((N_PAD, 4), jnp.float32)], axis=1)
    rowid = lax.broadcasted_iota(jnp.int32, (N_PAD, 1), 0)
    ys_ref[...] = jnp.where(rowid < N, y * dinv, 0.0)


def _tc3a_body(q_ref, ys_ref, dinv_ref, b3_ref, b6_ref, g1_ref, g2_ref):
    g = (q_ref[0] + q_ref[1] + ys_ref[...]) * dinv_ref[...]
    g1_ref[...] = g[:N, 0:2] + b3_ref[...]
    g2_ref[...] = g[:N, 2:4] + b6_ref[...]


def _tc3b_body(r1_ref, r2_ref, l1w_ref, l1b_ref, l2w_ref, l2b_ref,
               l3w_ref, l3b_ref, l4w_ref, l4b_ref, o1_ref, o2_ref):
    def mlp(r, w1, b1, w2, b2):
        hdn = jax.nn.relu(
            jnp.dot(r, w1, preferred_element_type=jnp.float32) + b1)
        return jnp.dot(hdn, w2, preferred_element_type=jnp.float32) + b2

    o1_ref[...] = mlp(r1_ref[...], l1w_ref[...], l1b_ref[...],
                      l2w_ref[...], l2b_ref[...])
    o2_ref[...] = mlp(r2_ref[...], l3w_ref[...], l3b_ref[...],
                      l4w_ref[...], l4b_ref[...])


def kernel(graph_state, graph_action, edge_index, W1, b1, W2, b2, W3, b3,
           L1_W, L1_b, L2_W, L2_b, W4, b4, W5, b5, W6, b6, L3_W, L3_b,
           L4_W, L4_b):
    f32 = jnp.float32
    s_pad = jnp.zeros((N_PAD, STATE_DIM), f32).at[:N].set(graph_state)
    a_pad = jnp.zeros((N_PAD, 1), f32).at[:N].set(graph_action)
    pad_idx = jnp.full((E_PAD - E,), N, jnp.int32)
    rowp = jnp.concatenate([edge_index[0], pad_idx]).reshape(CH, KC, 128)
    colp = jnp.concatenate([edge_index[1], pad_idx]).reshape(CH, KC, 128)
    w14 = jnp.concatenate([W1, W4], axis=1)
    ones1 = jnp.ones((128, FD), f32)
    z1 = jnp.zeros((N_PAD, FD), f32)
    z36 = jnp.zeros((N_PAD, F1), f32)
    z8 = jnp.zeros((N_PAD, 8), f32)

    degp = _deg(colp, ones1, z1)

    xs, dinv = pl.pallas_call(
        _tc1_body,
        out_shape=[jax.ShapeDtypeStruct((N_PAD, F1), f32),
                   jax.ShapeDtypeStruct((N_PAD, 1), f32)],
    )(s_pad, a_pad, w14, degp)

    p = _prop36(xs, rowp, colp, z36)

    ys = pl.pallas_call(
        _tc2_body,
        out_shape=jax.ShapeDtypeStruct((N_PAD, 8), f32),
    )(p, xs, dinv, W2, W5, W3, W6,
      b1.reshape(1, 16), b2.reshape(1, 16), b4.reshape(1, 16),
      b5.reshape(1, 16))

    q = _prop8(ys, rowp, colp, z8)

    g1, g2 = pl.pallas_call(
        _tc3a_body,
        out_shape=[jax.ShapeDtypeStruct((N, 2), f32),
                   jax.ShapeDtypeStruct((N, 2), f32)],
    )(q, ys, dinv, b3.reshape(1, 2), b6.reshape(1, 2))

    r1 = g1.reshape(N // NUM_AGENT, 2 * NUM_AGENT)
    r2 = g2.reshape(N // NUM_AGENT, 2 * NUM_AGENT)
    o1, o2 = pl.pallas_call(
        _tc3b_body,
        out_shape=[jax.ShapeDtypeStruct((N // NUM_AGENT, NUM_AGENT), f32),
                   jax.ShapeDtypeStruct((N // NUM_AGENT, NUM_AGENT), f32)],
    )(r1, r2, L1_W, L1_b.reshape(1, 32), L2_W, L2_b.reshape(1, NUM_AGENT),
      L3_W, L3_b.reshape(1, 32), L4_W, L4_b.reshape(1, NUM_AGENT))
    return (o1, o2)
